# Initial kernel scaffold; baseline (speedup 1.0000x reference)
import jax
import jax.numpy as jnp
from jax.experimental import pallas as pl


def kernel(X, A, graph_sizes, W1, b1, W2, b2, Wf, bf):
    raise NotImplementedError("write your pallas kernel here")


# speedup vs baseline: 14.4856x; 14.4856x over previous
import functools

import jax
import jax.numpy as jnp
from jax import lax
from jax.experimental import pallas as pl
from jax.experimental.pallas import tpu as pltpu
from jax.experimental.pallas import tpu_sc as plsc

_N = 10000
_E = 320000
_D = 128
_G = 10
_NC = 2
_NS = 16
_NW = _NC * _NS
_EPT = _E // _NW

_K = 80
_CH = _EPT // _K

_K3 = 80
_CH3 = _EPT // _K3
_NP = 10240
_UPT = _NP // _NS

_mesh = plsc.VectorSubcoreMesh(core_axis_name="c", subcore_axis_name="s")


@functools.partial(
    pl.kernel,
    out_type=jax.ShapeDtypeStruct((_NC * _N, _D), jnp.float32),
    mesh=_mesh,
    scratch_types=[
        pltpu.VMEM_SHARED((_N, _D), jnp.float32),
        pltpu.VMEM((_EPT,), jnp.int32),
        pltpu.VMEM((_EPT,), jnp.int32),
        pltpu.VMEM((_K, _D), jnp.float32),
        pltpu.VMEM((_K, _D), jnp.float32),
        pltpu.VMEM((_K,), jnp.int32),
        pltpu.SemaphoreType.DMA,
        pltpu.SemaphoreType.DMA,
    ],
)
def _sc_agg(h_hbm, src_hbm, dst_hbm, out_hbm,
            acc, srcs, dsts, rows0, rows1, idxb, sem0, sem1):
    cid = lax.axis_index("c")
    sid = lax.axis_index("s")
    wid = sid * _NC + cid
    ebase = wid * _EPT

    pltpu.sync_copy(src_hbm.at[pl.ds(ebase, _EPT)], srcs)
    pltpu.sync_copy(dst_hbm.at[pl.ds(ebase, _EPT)], dsts)

    def _zrow(i, c):
        for j in range(_D // 16):
            rows0[i, pl.ds(j * 16, 16)] = jnp.zeros((16,), jnp.float32)
        return c
    lax.fori_loop(0, _K, _zrow, 0)

    @pl.when(sid < _G)
    def _zero_acc():
        r0 = sid * (_N // _G)
        for k in range(12):
            pltpu.sync_copy(rows0, acc.at[pl.ds(r0 + k * _K, _K), :])
        pltpu.sync_copy(rows0.at[pl.ds(0, 40), :],
                        acc.at[pl.ds(r0 + 960, 40), :])
    plsc.subcore_barrier()

    def _start(ci, rows, sem):
        pltpu.async_copy(h_hbm.at[srcs.at[pl.ds(ci * _K, _K)]], rows, sem)

    def _finish(ci, rows, sem):
        pltpu.make_async_copy(h_hbm.at[srcs.at[pl.ds(0, _K)]], rows,
                              sem).wait()
        for g in range(_K // 16):
            idxb[pl.ds(g * 16, 16)] = dsts[pl.ds(ci * _K + g * 16, 16)]
        pltpu.sync_copy(rows, acc.at[idxb], add=True)

    _start(0, rows0, sem0)

    def _pair(g, c):
        i0 = 2 * g
        _start(i0 + 1, rows1, sem1)
        _finish(i0, rows0, sem0)
        _start(i0 + 2, rows0, sem0)
        _finish(i0 + 1, rows1, sem1)
        return c
    lax.fori_loop(0, _CH // 2, _pair, 0)
    _finish(_CH - 1, rows0, sem0)

    plsc.subcore_barrier()

    @pl.when(sid < _G)
    def _copy_out():
        o0 = sid * (_N // _G)
        pltpu.sync_copy(acc.at[pl.ds(o0, _N // _G), :],
                        out_hbm.at[pl.ds(cid * _N + o0, _N // _G), :])


@functools.partial(
    pl.kernel,
    out_type=jax.ShapeDtypeStruct((_NC * _NP,), jnp.float32),
    mesh=_mesh,
    scratch_types=[
        pltpu.VMEM_SHARED((_NP,), jnp.float32),
        pltpu.VMEM((_CH3, _K3), jnp.int32),
        pltpu.VMEM((_CH3, _K3), jnp.int32),
        pltpu.VMEM((_CH3, _K3), jnp.float32),
        pltpu.VMEM((_UPT,), jnp.float32),
        pltpu.SemaphoreType.DMA,
        pltpu.SemaphoreType.DMA,
    ],
)
def _sc_agg_scalar(z_hbm, src_hbm, dst_hbm, out_hbm,
                   acc, srcs, dsts, vals, zer, sem0, sem1):
    cid = lax.axis_index("c")
    sid = lax.axis_index("s")
    wid = sid * _NC + cid

    pltpu.sync_copy(src_hbm.at[wid], srcs)
    pltpu.sync_copy(dst_hbm.at[wid], dsts)

    def _z16(i, c):
        zer[pl.ds(i * 16, 16)] = jnp.zeros((16,), jnp.float32)
        return c
    lax.fori_loop(0, _UPT // 16, _z16, 0)
    u0 = sid * _UPT
    pltpu.sync_copy(zer, acc.at[pl.ds(u0, _UPT)])
    plsc.subcore_barrier()

    def _start(j, sem):
        pltpu.async_copy(z_hbm.at[srcs.at[j]], vals.at[j], sem)

    def _finish(j, sem):
        pltpu.make_async_copy(z_hbm.at[srcs.at[0]], vals.at[0], sem).wait()
        pltpu.sync_copy(vals.at[j], acc.at[dsts.at[j]], add=True)

    _start(0, sem0)

    def _pair(g, c):
        j0 = 2 * g
        _start(j0 + 1, sem1)
        _finish(j0, sem0)
        _start(j0 + 2, sem0)
        _finish(j0 + 1, sem1)
        return c
    lax.fori_loop(0, _CH3 // 2, _pair, 0)
    _finish(_CH3 - 1, sem0)

    plsc.subcore_barrier()
    pltpu.sync_copy(acc.at[pl.ds(u0, _UPT)],
                    out_hbm.at[pl.ds(cid * _NP + u0, _UPT)])


def _dense_relu(p, w, b):
    def body(p_ref, w_ref, b_ref, o_ref):
        agg = p_ref[0] + p_ref[1]
        o_ref[:] = jnp.maximum(
            jnp.dot(agg, w_ref[:], preferred_element_type=jnp.float32)
            + b_ref[:], 0.0)
    return pl.pallas_call(
        body,
        out_shape=jax.ShapeDtypeStruct((_N, _D), jnp.float32),
    )(p, w, b)


def _dense_relu_proj(p, w, b, wf):
    def body(p_ref, w_ref, b_ref, wf_ref, z_ref):
        agg = p_ref[0] + p_ref[1]
        h = jnp.maximum(
            jnp.dot(agg, w_ref[:], preferred_element_type=jnp.float32)
            + b_ref[:], 0.0)
        z_ref[:] = jnp.dot(h, wf_ref[:], preferred_element_type=jnp.float32)
    return pl.pallas_call(
        body,
        out_shape=jax.ShapeDtypeStruct((_N, 1), jnp.float32),
    )(p, w, b, wf)


def _readout(u_pad, gs, bf):
    def body(u_ref, gs_ref, bf_ref, o_ref):
        u = u_ref[0] + u_ref[1] + bf_ref[0]
        s = jax.nn.sigmoid(u)
        o_ref[:] = jnp.sum(s, axis=1) / gs_ref[:].astype(jnp.float32)
    return pl.pallas_call(
        body,
        out_shape=jax.ShapeDtypeStruct((_G,), jnp.float32),
        in_specs=[
            pl.BlockSpec(memory_space=pltpu.VMEM),
            pl.BlockSpec(memory_space=pltpu.VMEM),
            pl.BlockSpec(memory_space=pltpu.SMEM),
        ],
        out_specs=pl.BlockSpec(memory_space=pltpu.VMEM),
    )(u_pad, gs, bf)


def kernel(X, A, graph_sizes, W1, b1, W2, b2, Wf, bf):
    src = A[0]
    dst = A[1]
    src3 = src.reshape(_NW, _CH3, _K3)
    dst3 = dst.reshape(_NW, _CH3, _K3)

    p1 = _sc_agg(X, src, dst).reshape(_NC, _N, _D)
    h1 = _dense_relu(p1, W1, b1.reshape(1, _D))
    p2 = _sc_agg(h1, src, dst).reshape(_NC, _N, _D)
    z = _dense_relu_proj(p2, W2, b2.reshape(1, _D), Wf)
    u = _sc_agg_scalar(z.reshape(_N), src3, dst3).reshape(_NC, _NP)
    u2 = u[:, :_N].reshape(_NC, _G, _N // _G)
    u_pad = jnp.pad(u2, ((0, 0), (0, 0), (0, 24)), constant_values=-1e30)
    return _readout(u_pad, graph_sizes, bf)
